# baseline (speedup 1.0000x reference)
import functools

import jax
import jax.numpy as jnp
from jax import lax
from jax.experimental import pallas as pl
from jax.experimental.pallas import tpu as pltpu
from jax.experimental.pallas import tpu_sc as plsc

NUM_CORES = 2
NUM_SUBCORES = 16
NUM_WORKERS = NUM_CORES * NUM_SUBCORES
LANES = 16


def kernel(node_ids, node_tids, table):
    del node_tids
    B = node_ids.shape[0]
    D = table.shape[1]
    b_per_w = B // NUM_WORKERS
    mesh = plsc.VectorSubcoreMesh(core_axis_name="c", subcore_axis_name="s")

    @functools.partial(
        pl.kernel,
        mesh=mesh,
        out_type=jax.ShapeDtypeStruct((B * D,), table.dtype),
        scratch_types=[
            pltpu.VMEM((b_per_w,), jnp.int32),
            pltpu.VMEM((b_per_w, D), table.dtype),
            pltpu.VMEM((b_per_w * D,), table.dtype),
            pltpu.SemaphoreType.DMA,
        ],
        compiler_params=pltpu.CompilerParams(use_tc_tiling_on_sc=False),
    )
    def gather_kernel(table_hbm, idx_hbm, out_hbm, idx_v, rows_v, flat_v,
                      sem):
        wid = lax.axis_index("s") * NUM_CORES + lax.axis_index("c")
        base = wid * b_per_w
        pltpu.sync_copy(idx_hbm.at[pl.ds(base, b_per_w)], idx_v)
        pltpu.async_copy(table_hbm.at[idx_v], rows_v, sem).wait()

        def copy_rows(c, carry):
            for r in range(LANES):
                i = c * LANES + r
                for d0 in range(D // LANES):
                    flat_v[pl.ds(i * D + d0 * LANES, LANES)] = (
                        rows_v[i, pl.ds(d0 * LANES, LANES)]
                    )
            return carry

        lax.fori_loop(0, b_per_w // LANES, copy_rows, 0)
        pltpu.sync_copy(flat_v, out_hbm.at[pl.ds(base * D, b_per_w * D)])

    return gather_kernel(table, node_ids).reshape(B, D)
